# Initial kernel scaffold; baseline (speedup 1.0000x reference)
import jax
import jax.numpy as jnp
from jax.experimental import pallas as pl


def kernel(x, w_ggl, b_ggl, c1w1, c1w2a, c1w2b, c1w3a, c1w3b, c1w3c, c1b1, c1b2, c1b3, c2w1, c2w2a, c2w2b, c2w3a, c2w3b, c2w3c, c2b1, c2b2, c2b3, bn1_g, bn1_b, bn2_g, bn2_b, w5, b5):
    raise NotImplementedError("write your pallas kernel here")


# speedup vs baseline: 35.2036x; 35.2036x over previous
import jax
import jax.numpy as jnp
from jax.experimental import pallas as pl
from jax.experimental.pallas import tpu as pltpu

_N = 512
_EPS = 1e-5


def _mm(a, b):
    return jax.lax.dot_general(a, b, (((1,), (0,)), ((), ())),
                               preferred_element_type=jnp.float32)


def _fused(x_ref, wggl_ref, bggl_ref,
           c1w1_ref, c1w2a_ref, c1w2b_ref, c1w3a_ref, c1w3b_ref, c1w3c_ref,
           c1b1_ref, c1b2_ref, c1b3_ref,
           c2w1_ref, c2w2a_ref, c2w2b_ref, c2w3a_ref, c2w3b_ref, c2w3c_ref,
           c2b1_ref, c2b2_ref, c2b3_ref,
           bn1g_ref, bn1b_ref, bn2g_ref, bn2b_ref,
           w5_ref, b5_ref, out_ref):
    n = _N
    x2 = x_ref[...]
    atrr = jax.nn.sigmoid(_mm(x2, wggl_ref[...]) + bggl_ref[...])

    a = jax.lax.dot_general(atrr, atrr, (((1,), (1,)), ((), ())),
                            preferred_element_type=jnp.float32)
    maxval = jnp.max(a, axis=1, keepdims=True)
    row = jax.lax.broadcasted_iota(jnp.int32, (n, n), 0)
    col = jax.lax.broadcasted_iota(jnp.int32, (n, n), 1)
    offdiag = (row != col).astype(jnp.float32)

    inv_max = 1.0 / maxval
    adj = a * offdiag * jnp.transpose(inv_max)
    deg = jnp.sum(adj, axis=1, keepdims=True)
    dis = jnp.where(deg > 0.0, jax.lax.rsqrt(deg), 0.0)
    lhatT = (a * offdiag) * (-(dis * inv_max)) * jnp.transpose(dis)

    t1 = _mm(lhatT, x2)
    t2 = 2.0 * _mm(lhatT, t1) - x2
    h1 = _mm(x2, c1w1_ref[...]) + c1b1_ref[...]
    h2 = _mm(x2, c1w2a_ref[...]) + _mm(t1, c1w2b_ref[...]) + c1b2_ref[...]
    h3 = (_mm(x2, c1w3a_ref[...]) + _mm(t1, c1w3b_ref[...])
          + _mm(t2, c1w3c_ref[...]) + c1b3_ref[...])
    h = jnp.concatenate([h1, h2, h3], axis=1)

    mu = jnp.mean(h, axis=0, keepdims=True)
    var = jnp.mean(h * h, axis=0, keepdims=True) - mu * mu
    h = (h - mu) * jax.lax.rsqrt(var + _EPS) * bn1g_ref[...] + bn1b_ref[...]

    s1 = _mm(lhatT, h)
    s2 = 2.0 * _mm(lhatT, s1) - h
    g1 = _mm(h, c2w1_ref[...]) + c2b1_ref[...]
    g2 = _mm(h, c2w2a_ref[...]) + _mm(s1, c2w2b_ref[...]) + c2b2_ref[...]
    g3 = (_mm(h, c2w3a_ref[...]) + _mm(s1, c2w3b_ref[...])
          + _mm(s2, c2w3c_ref[...]) + c2b3_ref[...])
    g = jnp.concatenate([g1, g2, g3], axis=1)

    mu2 = jnp.mean(g, axis=0, keepdims=True)
    var2 = jnp.mean(g * g, axis=0, keepdims=True) - mu2 * mu2
    g = (g - mu2) * jax.lax.rsqrt(var2 + _EPS) * bn2g_ref[...] + bn2b_ref[...]

    out_ref[...] = jnp.maximum(_mm(g, w5_ref[...]) + b5_ref[...], 0.0)


@jax.jit
def kernel(x, w_ggl, b_ggl, c1w1, c1w2a, c1w2b, c1w3a, c1w3b, c1w3c,
           c1b1, c1b2, c1b3, c2w1, c2w2a, c2w2b, c2w3a, c2w3b, c2w3c,
           c2b1, c2b2, c2b3, bn1_g, bn1_b, bn2_g, bn2_b, w5, b5):
    args = [x, w_ggl, b_ggl.reshape(1, -1),
            c1w1, c1w2a, c1w2b, c1w3a, c1w3b, c1w3c,
            c1b1.reshape(1, -1), c1b2.reshape(1, -1), c1b3.reshape(1, -1),
            c2w1, c2w2a, c2w2b, c2w3a, c2w3b, c2w3c,
            c2b1.reshape(1, -1), c2b2.reshape(1, -1), c2b3.reshape(1, -1),
            bn1_g.reshape(1, -1), bn1_b.reshape(1, -1),
            bn2_g.reshape(1, -1), bn2_b.reshape(1, -1),
            w5, b5.reshape(1, -1)]
    return pl.pallas_call(
        _fused,
        out_shape=jax.ShapeDtypeStruct((_N, 256), jnp.float32),
    )(*args)


# speedup vs baseline: 38.2661x; 1.0870x over previous
import jax
import jax.numpy as jnp
from jax.experimental import pallas as pl
from jax.experimental.pallas import tpu as pltpu

_N = 512
_EPS = 1e-5


def _mm(a, b):
    return jax.lax.dot_general(a, b, (((1,), (0,)), ((), ())),
                               preferred_element_type=jnp.float32)


def _fused(x_ref, wggl_ref, bggl_ref,
           c1w1_ref, c1w2a_ref, c1w2b_ref, c1w3a_ref, c1w3b_ref, c1w3c_ref,
           c1b1_ref, c1b2_ref, c1b3_ref,
           c2w1_ref, c2w2a_ref, c2w2b_ref, c2w3a_ref, c2w3b_ref, c2w3c_ref,
           c2b1_ref, c2b2_ref, c2b3_ref,
           bn1g_ref, bn1b_ref, bn2g_ref, bn2b_ref,
           w5_ref, b5_ref, out_ref):
    n = _N
    x2 = x_ref[...]
    atrr = jax.nn.sigmoid(_mm(x2, wggl_ref[...]) + bggl_ref[...])

    a = jax.lax.dot_general(atrr, atrr, (((1,), (1,)), ((), ())),
                            preferred_element_type=jnp.float32)
    maxval = jnp.max(a, axis=1, keepdims=True)
    row = jax.lax.broadcasted_iota(jnp.int32, (n, n), 0)
    col = jax.lax.broadcasted_iota(jnp.int32, (n, n), 1)
    offdiag = (row != col).astype(jnp.float32)

    inv_max = 1.0 / maxval
    adj = a * offdiag * jnp.transpose(inv_max)
    deg = jnp.sum(adj, axis=1, keepdims=True)
    dis = jnp.where(deg > 0.0, jax.lax.rsqrt(deg), 0.0)
    lhatT = (a * offdiag) * (-(dis * inv_max)) * jnp.transpose(dis)

    t1 = _mm(lhatT, x2)
    t2 = 2.0 * _mm(lhatT, t1) - x2
    h1 = _mm(x2, c1w1_ref[...]) + c1b1_ref[...]
    h2 = _mm(x2, c1w2a_ref[...]) + _mm(t1, c1w2b_ref[...]) + c1b2_ref[...]
    h3 = (_mm(x2, c1w3a_ref[...]) + _mm(t1, c1w3b_ref[...])
          + _mm(t2, c1w3c_ref[...]) + c1b3_ref[...])
    h = jnp.concatenate([h1, h2, h3], axis=1)

    mu = jnp.mean(h, axis=0, keepdims=True)
    var = jnp.mean(h * h, axis=0, keepdims=True) - mu * mu
    h = (h - mu) * jax.lax.rsqrt(var + _EPS) * bn1g_ref[...] + bn1b_ref[...]

    s1 = _mm(lhatT, h)
    s2 = 2.0 * _mm(lhatT, s1) - h
    g1 = _mm(h, c2w1_ref[...]) + c2b1_ref[...]
    g2 = _mm(h, c2w2a_ref[...]) + _mm(s1, c2w2b_ref[...]) + c2b2_ref[...]
    g3 = (_mm(h, c2w3a_ref[...]) + _mm(s1, c2w3b_ref[...])
          + _mm(s2, c2w3c_ref[...]) + c2b3_ref[...])
    g = jnp.concatenate([g1, g2, g3], axis=1)

    mu2 = jnp.mean(g, axis=0, keepdims=True)
    var2 = jnp.mean(g * g, axis=0, keepdims=True) - mu2 * mu2
    g = (g - mu2) * jax.lax.rsqrt(var2 + _EPS) * bn2g_ref[...] + bn2b_ref[...]

    out_ref[...] = jnp.maximum(_mm(g, w5_ref[...]) + b5_ref[...], 0.0)


@jax.jit
def kernel(x, w_ggl, b_ggl, c1w1, c1w2a, c1w2b, c1w3a, c1w3b, c1w3c,
           c1b1, c1b2, c1b3, c2w1, c2w2a, c2w2b, c2w3a, c2w3b, c2w3c,
           c2b1, c2b2, c2b3, bn1_g, bn1_b, bn2_g, bn2_b, w5, b5):
    return pl.pallas_call(
        _fused,
        out_shape=jax.ShapeDtypeStruct((_N, 256), jnp.float32),
    )(x, w_ggl, b_ggl, c1w1, c1w2a, c1w2b, c1w3a, c1w3b, c1w3c,
      c1b1, c1b2, c1b3, c2w1, c2w2a, c2w2b, c2w3a, c2w3b, c2w3c,
      c2b1, c2b2, c2b3, bn1_g, bn1_b, bn2_g, bn2_b, w5, b5)


# speedup vs baseline: 93.4748x; 2.4428x over previous
import jax
import jax.numpy as jnp
from jax.experimental import pallas as pl
from jax.experimental.pallas import tpu as pltpu

_N = 512
_EPS = 1e-5


def _mm(a, b):
    return jax.lax.dot_general(a, b, (((1,), (0,)), ((), ())),
                               preferred_element_type=jnp.float32)


def _mmt(a, bt):
    return jax.lax.dot_general(a, bt, (((1,), (1,)), ((), ())),
                               preferred_element_type=jnp.float32)


def _fused(x_ref, wggl_ref, bggl_ref,
           c1w1_ref, c1w2a_ref, c1w2b_ref, c1w3a_ref, c1w3b_ref, c1w3c_ref,
           c1b1_ref, c1b2_ref, c1b3_ref,
           c2w1_ref, c2w2a_ref, c2w2b_ref, c2w3a_ref, c2w3b_ref, c2w3c_ref,
           c2b1_ref, c2b2_ref, c2b3_ref,
           bn1g_ref, bn1b_ref, bn2g_ref, bn2b_ref,
           w5_ref, b5_ref, out_ref):
    n = _N
    x2 = x_ref[...]
    atrr = jax.nn.sigmoid(_mmt(x2, wggl_ref[...]) + bggl_ref[...])

    a = jax.lax.dot_general(atrr, atrr, (((1,), (1,)), ((), ())),
                            preferred_element_type=jnp.float32)
    maxval = jnp.max(a, axis=1, keepdims=True)
    row = jax.lax.broadcasted_iota(jnp.int32, (n, n), 0)
    col = jax.lax.broadcasted_iota(jnp.int32, (n, n), 1)
    offdiag = (row != col).astype(jnp.float32)

    inv_max = 1.0 / maxval
    adj = a * offdiag * jnp.transpose(inv_max)
    deg = jnp.sum(adj, axis=1, keepdims=True)
    dis = jnp.where(deg > 0.0, jax.lax.rsqrt(deg), 0.0)
    lhatT = (a * offdiag) * (-(dis * inv_max)) * jnp.transpose(dis)

    t1 = _mm(lhatT, x2)
    t2 = 2.0 * _mm(lhatT, t1) - x2
    h1 = _mmt(x2, c1w1_ref[...]) + c1b1_ref[...]
    h2 = _mmt(x2, c1w2a_ref[...]) + _mmt(t1, c1w2b_ref[...]) + c1b2_ref[...]
    h3 = (_mmt(x2, c1w3a_ref[...]) + _mmt(t1, c1w3b_ref[...])
          + _mmt(t2, c1w3c_ref[...]) + c1b3_ref[...])
    h = jnp.concatenate([h1, h2, h3], axis=1)

    mu = jnp.mean(h, axis=0, keepdims=True)
    var = jnp.mean(h * h, axis=0, keepdims=True) - mu * mu
    h = (h - mu) * jax.lax.rsqrt(var + _EPS) * bn1g_ref[...] + bn1b_ref[...]

    s1 = _mm(lhatT, h)
    s2 = 2.0 * _mm(lhatT, s1) - h
    g1 = _mmt(h, c2w1_ref[...]) + c2b1_ref[...]
    g2 = _mmt(h, c2w2a_ref[...]) + _mmt(s1, c2w2b_ref[...]) + c2b2_ref[...]
    g3 = (_mmt(h, c2w3a_ref[...]) + _mmt(s1, c2w3b_ref[...])
          + _mmt(s2, c2w3c_ref[...]) + c2b3_ref[...])
    g = jnp.concatenate([g1, g2, g3], axis=1)

    mu2 = jnp.mean(g, axis=0, keepdims=True)
    var2 = jnp.mean(g * g, axis=0, keepdims=True) - mu2 * mu2
    g = (g - mu2) * jax.lax.rsqrt(var2 + _EPS) * bn2g_ref[...] + bn2b_ref[...]

    out_ref[...] = jnp.maximum(_mm(g, w5_ref[...]) + b5_ref[...], 0.0)


@jax.jit
def kernel(x, w_ggl, b_ggl, c1w1, c1w2a, c1w2b, c1w3a, c1w3b, c1w3c,
           c1b1, c1b2, c1b3, c2w1, c2w2a, c2w2b, c2w3a, c2w3b, c2w3c,
           c2b1, c2b2, c2b3, bn1_g, bn1_b, bn2_g, bn2_b, w5, b5):
    return pl.pallas_call(
        _fused,
        out_shape=jax.ShapeDtypeStruct((_N, 256), jnp.float32),
    )(x, w_ggl.T, b_ggl,
      c1w1.T, c1w2a.T, c1w2b.T, c1w3a.T, c1w3b.T, c1w3c.T,
      c1b1, c1b2, c1b3,
      c2w1.T, c2w2a.T, c2w2b.T, c2w3a.T, c2w3b.T, c2w3c.T,
      c2b1, c2b2, c2b3, bn1_g, bn1_b, bn2_g, bn2_b, w5, b5)


# speedup vs baseline: 100.7663x; 1.0780x over previous
import jax
import jax.numpy as jnp
from jax.experimental import pallas as pl
from jax.experimental.pallas import tpu as pltpu

_N = 512
_EPS = 1e-5


def _mm(a, b):
    return jax.lax.dot_general(a, b, (((1,), (0,)), ((), ())),
                               preferred_element_type=jnp.float32)


def _mmt(a, bt):
    return jax.lax.dot_general(a, bt, (((1,), (1,)), ((), ())),
                               preferred_element_type=jnp.float32)


def _fused(x_ref, wggl_ref, bggl_ref,
           c1w1_ref, c1w2a_ref, c1w2b_ref, c1w3a_ref, c1w3b_ref, c1w3c_ref,
           c1b1_ref, c1b2_ref, c1b3_ref,
           c2w1_ref, c2w2a_ref, c2w2b_ref, c2w3a_ref, c2w3b_ref, c2w3c_ref,
           c2b1_ref, c2b2_ref, c2b3_ref,
           bn1g_ref, bn1b_ref, bn2g_ref, bn2b_ref,
           w5_ref, b5_ref, out_ref):
    n = _N
    x2 = x_ref[...]
    atrr = jax.nn.sigmoid(_mmt(x2, wggl_ref[...]) + bggl_ref[...])

    a = jax.lax.dot_general(atrr, atrr, (((1,), (1,)), ((), ())),
                            preferred_element_type=jnp.float32)
    maxval = jnp.max(a, axis=1, keepdims=True)
    row = jax.lax.broadcasted_iota(jnp.int32, (n, n), 0)
    col = jax.lax.broadcasted_iota(jnp.int32, (n, n), 1)
    offdiag = (row != col).astype(jnp.float32)

    inv_max = 1.0 / maxval
    adj = a * offdiag * jnp.transpose(inv_max)
    deg = jnp.sum(adj, axis=1, keepdims=True)
    dis = jnp.where(deg > 0.0, jax.lax.rsqrt(deg), 0.0)
    lhatT = (a * offdiag) * (-(dis * inv_max)) * jnp.transpose(dis)

    t1 = _mm(lhatT, x2)
    t2 = 2.0 * _mm(lhatT, t1) - x2
    h1 = _mmt(x2, c1w1_ref[...]) + c1b1_ref[...]
    h2 = _mmt(x2, c1w2a_ref[...]) + _mmt(t1, c1w2b_ref[...]) + c1b2_ref[...]
    h3 = (_mmt(x2, c1w3a_ref[...]) + _mmt(t1, c1w3b_ref[...])
          + _mmt(t2, c1w3c_ref[...]) + c1b3_ref[...])
    h = jnp.concatenate([h1, h2, h3], axis=1)

    mu = jnp.mean(h, axis=0, keepdims=True)
    var = jnp.mean(h * h, axis=0, keepdims=True) - mu * mu
    h = (h - mu) * jax.lax.rsqrt(var + _EPS) * bn1g_ref[...] + bn1b_ref[...]

    g1 = _mmt(h, c2w1_ref[...]) + c2b1_ref[...]
    u2 = _mmt(h, c2w2b_ref[...])
    g2 = _mmt(h, c2w2a_ref[...]) + _mm(lhatT, u2) + c2b2_ref[...]
    u3 = _mmt(h, c2w3b_ref[...])
    p3 = _mmt(h, c2w3c_ref[...])
    q3 = _mm(lhatT, p3)
    g3 = (_mmt(h, c2w3a_ref[...]) + _mm(lhatT, u3)
          + 2.0 * _mm(lhatT, q3) - p3 + c2b3_ref[...])
    g = jnp.concatenate([g1, g2, g3], axis=1)

    mu2 = jnp.mean(g, axis=0, keepdims=True)
    var2 = jnp.mean(g * g, axis=0, keepdims=True) - mu2 * mu2
    g = (g - mu2) * jax.lax.rsqrt(var2 + _EPS) * bn2g_ref[...] + bn2b_ref[...]

    out_ref[...] = jnp.maximum(_mm(g, w5_ref[...]) + b5_ref[...], 0.0)


@jax.jit
def kernel(x, w_ggl, b_ggl, c1w1, c1w2a, c1w2b, c1w3a, c1w3b, c1w3c,
           c1b1, c1b2, c1b3, c2w1, c2w2a, c2w2b, c2w3a, c2w3b, c2w3c,
           c2b1, c2b2, c2b3, bn1_g, bn1_b, bn2_g, bn2_b, w5, b5):
    return pl.pallas_call(
        _fused,
        out_shape=jax.ShapeDtypeStruct((_N, 256), jnp.float32),
    )(x, w_ggl.T, b_ggl,
      c1w1.T, c1w2a.T, c1w2b.T, c1w3a.T, c1w3b.T, c1w3c.T,
      c1b1, c1b2, c1b3,
      c2w1.T, c2w2a.T, c2w2b.T, c2w3a.T, c2w3b.T, c2w3c.T,
      c2b1, c2b2, c2b3, bn1_g, bn1_b, bn2_g, bn2_b, w5, b5)


# speedup vs baseline: 105.4227x; 1.0462x over previous
import jax
import jax.numpy as jnp
from jax.experimental import pallas as pl
from jax.experimental.pallas import tpu as pltpu

_N = 512
_EPS = 1e-5


def _mm(a, b):
    return jax.lax.dot_general(a, b, (((1,), (0,)), ((), ())),
                               preferred_element_type=jnp.float32)


def _mmt(a, bt):
    return jax.lax.dot_general(a, bt, (((1,), (1,)), ((), ())),
                               preferred_element_type=jnp.float32)


def _fused(x_ref, wggl_ref, bggl_ref,
           c1w1_h, c1w2a_h, c1w2b_h, c1w3a_h, c1w3b_h, c1w3c_h,
           c1b1_ref, c1b2_ref, c1b3_ref,
           c2w1_h, c2w2a_h, c2w2b_h, c2w3a_h, c2w3b_h, c2w3c_h,
           c2b1_ref, c2b2_ref, c2b3_ref,
           bn1g_ref, bn1b_ref, bn2g_ref, bn2b_ref,
           w5_h, b5_ref, out_ref,
           c1w1_v, c1w2a_v, c1w2b_v, c1w3a_v, c1w3b_v, c1w3c_v,
           c2w1_v, c2w2a_v, c2w2b_v, c2w3a_v, c2w3b_v, c2w3c_v,
           w5_v, sem):
    n = _N
    srcs = [c1w1_h, c1w2a_h, c1w2b_h, c1w3a_h, c1w3b_h, c1w3c_h,
            c2w1_h, c2w2a_h, c2w2b_h, c2w3a_h, c2w3b_h, c2w3c_h, w5_h]
    dsts = [c1w1_v, c1w2a_v, c1w2b_v, c1w3a_v, c1w3b_v, c1w3c_v,
            c2w1_v, c2w2a_v, c2w2b_v, c2w3a_v, c2w3b_v, c2w3c_v, w5_v]
    copies = [pltpu.make_async_copy(s, d, sem.at[i])
              for i, (s, d) in enumerate(zip(srcs, dsts))]
    for cp in copies:
        cp.start()

    x2 = x_ref[...]
    atrr = jax.nn.sigmoid(_mmt(x2, wggl_ref[...]) + bggl_ref[...])
    a = jax.lax.dot_general(atrr, atrr, (((1,), (1,)), ((), ())),
                            preferred_element_type=jnp.float32)
    maxval = jnp.max(a, axis=1, keepdims=True)
    row = jax.lax.broadcasted_iota(jnp.int32, (n, n), 0)
    col = jax.lax.broadcasted_iota(jnp.int32, (n, n), 1)
    offdiag = (row != col).astype(jnp.float32)
    inv_max = 1.0 / maxval
    adj = a * offdiag * jnp.transpose(inv_max)
    deg = jnp.sum(adj, axis=1, keepdims=True)
    dis = jnp.where(deg > 0.0, jax.lax.rsqrt(deg), 0.0)
    lhatT = (a * offdiag) * (-(dis * inv_max)) * jnp.transpose(dis)

    t1 = _mm(lhatT, x2)
    t2 = 2.0 * _mm(lhatT, t1) - x2

    for cp in copies[:6]:
        cp.wait()
    h1 = _mmt(x2, c1w1_v[...]) + c1b1_ref[...]
    h2 = _mmt(x2, c1w2a_v[...]) + _mmt(t1, c1w2b_v[...]) + c1b2_ref[...]
    h3 = (_mmt(x2, c1w3a_v[...]) + _mmt(t1, c1w3b_v[...])
          + _mmt(t2, c1w3c_v[...]) + c1b3_ref[...])
    h = jnp.concatenate([h1, h2, h3], axis=1)

    mu = jnp.mean(h, axis=0, keepdims=True)
    var = jnp.mean(h * h, axis=0, keepdims=True) - mu * mu
    h = (h - mu) * jax.lax.rsqrt(var + _EPS) * bn1g_ref[...] + bn1b_ref[...]

    for cp in copies[6:12]:
        cp.wait()
    g1 = _mmt(h, c2w1_v[...]) + c2b1_ref[...]
    u2 = _mmt(h, c2w2b_v[...])
    g2 = _mmt(h, c2w2a_v[...]) + _mm(lhatT, u2) + c2b2_ref[...]
    u3 = _mmt(h, c2w3b_v[...])
    p3 = _mmt(h, c2w3c_v[...])
    q3 = _mm(lhatT, p3)
    g3 = (_mmt(h, c2w3a_v[...]) + _mm(lhatT, u3)
          + 2.0 * _mm(lhatT, q3) - p3 + c2b3_ref[...])
    g = jnp.concatenate([g1, g2, g3], axis=1)

    mu2 = jnp.mean(g, axis=0, keepdims=True)
    var2 = jnp.mean(g * g, axis=0, keepdims=True) - mu2 * mu2
    g = (g - mu2) * jax.lax.rsqrt(var2 + _EPS) * bn2g_ref[...] + bn2b_ref[...]

    copies[12].wait()
    out_ref[...] = jnp.maximum(_mm(g, w5_v[...]) + b5_ref[...], 0.0)


_HBM = pl.BlockSpec(memory_space=pl.ANY)
_VMEM = pl.BlockSpec(memory_space=pltpu.MemorySpace.VMEM)


@jax.jit
def kernel(x, w_ggl, b_ggl, c1w1, c1w2a, c1w2b, c1w3a, c1w3b, c1w3c,
           c1b1, c1b2, c1b3, c2w1, c2w2a, c2w2b, c2w3a, c2w3b, c2w3c,
           c2b1, c2b2, c2b3, bn1_g, bn1_b, bn2_g, bn2_b, w5, b5):
    f32 = jnp.float32
    return pl.pallas_call(
        _fused,
        out_shape=jax.ShapeDtypeStruct((_N, 256), f32),
        in_specs=[_VMEM, _VMEM, _VMEM,
                  _HBM, _HBM, _HBM, _HBM, _HBM, _HBM,
                  _VMEM, _VMEM, _VMEM,
                  _HBM, _HBM, _HBM, _HBM, _HBM, _HBM,
                  _VMEM, _VMEM, _VMEM,
                  _VMEM, _VMEM, _VMEM, _VMEM,
                  _HBM, _VMEM],
        scratch_shapes=(
            [pltpu.VMEM((400, 256), f32)] * 6
            + [pltpu.VMEM((100, 1200), f32)] * 6
            + [pltpu.VMEM((300, 256), f32),
               pltpu.SemaphoreType.DMA((13,))]),
    )(x, w_ggl.T, b_ggl,
      c1w1.T, c1w2a.T, c1w2b.T, c1w3a.T, c1w3b.T, c1w3c.T,
      c1b1, c1b2, c1b3,
      c2w1.T, c2w2a.T, c2w2b.T, c2w3a.T, c2w3b.T, c2w3c.T,
      c2b1, c2b2, c2b3, bn1_g, bn1_b, bn2_g, bn2_b, w5, b5)
